# baseline (speedup 1.0000x reference)
import jax
import jax.numpy as jnp
from jax.experimental import pallas as pl
from jax.experimental.pallas import tpu as pltpu

D_MODEL = 1024
D_SPACE = 64
S = 2048
N_OUT = 20480
TN = 512
NUM_J = N_OUT // TN
XK = D_MODEL // 2

_SEGS = (
    (7, 24, 8, 32),
    (4, 8, 4, 20),
    (5, 8, 4, 24),
    (6, 12, 4, 28),
    (0, 0, 4, 0),
    (1, 0, 4, 4),
    (2, 4, 4, 8),
    (3, 16, 8, 12),
)
_HTAB = tuple(h for h, n0, nn, o0 in _SEGS for _ in range(nn))
_NTAB = tuple(n0 + k for h, n0, nn, o0 in _SEGS for k in range(nn))
_OTAB = tuple(o0 + k for h, n0, nn, o0 in _SEGS for k in range(nn))


def _ln_into(scr, k, t, g_ref, b_ref):
    g = g_ref[:, k * D_SPACE:(k + 1) * D_SPACE]
    b = b_ref[:, k * D_SPACE:(k + 1) * D_SPACE]
    m = jnp.mean(t, axis=-1, keepdims=True)
    v = jnp.mean((t - m) ** 2, axis=-1, keepdims=True)
    scr[k] = ((t - m) * jax.lax.rsqrt(v + 1e-5) * g + b).astype(jnp.bfloat16)


def _body(tab_ref, x_ref, ca_ref, ck_ref, ne_ref, Wx_ref, bx_ref, Wr_ref,
          br_ref, Wkn_ref, bkn_ref, g_ref, beta_ref, out_ref, h_scr, px_scr):
    s = pl.program_id(0)

    @pl.when(s == 0)
    def _know_prologue():
        pk = jnp.dot(ck_ref[...], Wkn_ref[...],
                     preferred_element_type=jnp.float32) + bkn_ref[...]
        _ln_into(h_scr, 7, pk, g_ref, beta_ref)

    @pl.when(s == 1)
    def _attn_prologue():
        pr = jnp.dot(ca_ref[...], Wr_ref[...],
                     preferred_element_type=jnp.float32) + br_ref[...]
        for k in range(3):
            _ln_into(h_scr, 4 + k, pr[:, k * D_SPACE:(k + 1) * D_SPACE],
                     g_ref, beta_ref)

    @pl.when(s == 2)
    def _x_prologue_a():
        px_scr[...] = jnp.dot(x_ref[...], Wx_ref[0],
                              preferred_element_type=jnp.float32)

    @pl.when(s == 3)
    def _x_prologue_b():
        px = px_scr[...] + jnp.dot(x_ref[...], Wx_ref[1],
                                   preferred_element_type=jnp.float32)
        px = px + bx_ref[...]
        for k in range(4):
            _ln_into(h_scr, k, px[:, k * D_SPACE:(k + 1) * D_SPACE],
                     g_ref, beta_ref)

    e = ne_ref[...].astype(jnp.float32)
    inv = 1.0 / jnp.maximum(
        jnp.sqrt(jnp.sum(e * e, axis=-1, keepdims=True)), 1e-12)
    en = (e * inv).astype(jnp.bfloat16)
    h = h_scr[tab_ref[2, s]]
    out_ref[...] = jax.lax.dot_general(
        h, en, (((1,), (1,)), ((), ())), preferred_element_type=jnp.float32)


def kernel(x, ctx_attn, ctx_know, neuron_emb, W_feat, b_feat, W_know, b_know,
           W_rQ, b_rQ, W_rK, b_rK, W_rV, b_rV, W_rKn, b_rKn,
           g_fqkQ, beta_fqkQ, g_fqkK, beta_fqkK, g_fv, beta_fv,
           g_fkn, beta_fkn, g_rQ, beta_rQ, g_rK, beta_rK,
           g_rV, beta_rV, g_rKn, beta_rKn):
    B = x.shape[0]
    x2 = x.reshape(B * S, D_MODEL).astype(jnp.bfloat16)
    ca = ctx_attn.reshape(B * S, -1).astype(jnp.bfloat16)
    ck = ctx_know.reshape(B * S, -1).astype(jnp.bfloat16)

    Wx = jnp.concatenate([W_feat, W_know], axis=1)
    Wxs = Wx.astype(jnp.bfloat16).reshape(2, XK, 256)
    bx = jnp.concatenate([b_feat, b_know])[None, :]
    Wr = jnp.concatenate([W_rQ, W_rK, W_rV], axis=1).astype(jnp.bfloat16)
    br = jnp.concatenate([b_rQ, b_rK, b_rV])[None, :]
    Wkn = W_rKn.astype(jnp.bfloat16)
    bkn = b_rKn[None, :]
    g = jnp.concatenate([g_fqkQ, g_fqkK, g_fv, g_fkn,
                         g_rQ, g_rK, g_rV, g_rKn])[None, :]
    beta = jnp.concatenate([beta_fqkQ, beta_fqkK, beta_fv, beta_fkn,
                            beta_rQ, beta_rK, beta_rV, beta_rKn])[None, :]

    tab = jnp.asarray([_NTAB, _OTAB, _HTAB], dtype=jnp.int32)
    full = lambda a: pl.BlockSpec(a.shape, lambda s, t: (0,) * a.ndim)

    grid_spec = pltpu.PrefetchScalarGridSpec(
        num_scalar_prefetch=1,
        grid=(NUM_J,),
        in_specs=[
            pl.BlockSpec((B * S, XK),
                         lambda s, t: (0, jnp.clip(s - 2, 0, 1))),
            full(ca), full(ck),
            pl.BlockSpec((TN, D_SPACE), lambda s, t: (t[0, s], 0)),
            full(Wxs), full(bx), full(Wr), full(br),
            full(Wkn), full(bkn), full(g), full(beta),
        ],
        out_specs=pl.BlockSpec((B * S, TN), lambda s, t: (0, t[1, s])),
        scratch_shapes=[pltpu.VMEM((8, B * S, D_SPACE), jnp.bfloat16),
                        pltpu.VMEM((B * S, 256), jnp.float32)],
    )

    out = pl.pallas_call(
        _body,
        grid_spec=grid_spec,
        out_shape=jax.ShapeDtypeStruct((B * S, N_OUT), jnp.float32),
    )(tab, x2, ca, ck, neuron_emb.astype(jnp.bfloat16),
      Wxs, bx, Wr, br, Wkn, bkn, g, beta)

    return out.reshape(B, S, N_OUT)


# speedup vs baseline: 1.1158x; 1.1158x over previous
import jax
import jax.numpy as jnp
from jax.experimental import pallas as pl
from jax.experimental.pallas import tpu as pltpu

D_MODEL = 1024
D_SPACE = 64
S = 2048
N_OUT = 20480
TN = 1024
NUM_J = N_OUT // TN
XK = D_MODEL // 2

_SEGS = (
    (7, 12, 4, 16),
    (4, 4, 2, 10),
    (5, 4, 2, 12),
    (6, 6, 2, 14),
    (0, 0, 2, 0),
    (1, 0, 2, 2),
    (2, 2, 2, 4),
    (3, 8, 4, 6),
)
_HTAB = tuple(h for h, n0, nn, o0 in _SEGS for _ in range(nn))
_NTAB = tuple(n0 + k for h, n0, nn, o0 in _SEGS for k in range(nn))
_OTAB = tuple(o0 + k for h, n0, nn, o0 in _SEGS for k in range(nn))


def _ln_into(scr, k, t, g_ref, b_ref):
    g = g_ref[:, k * D_SPACE:(k + 1) * D_SPACE]
    b = b_ref[:, k * D_SPACE:(k + 1) * D_SPACE]
    m = jnp.mean(t, axis=-1, keepdims=True)
    v = jnp.mean((t - m) ** 2, axis=-1, keepdims=True)
    scr[k] = ((t - m) * jax.lax.rsqrt(v + 1e-5) * g + b).astype(jnp.bfloat16)


def _body(tab_ref, x_ref, ca_ref, ck_ref, ne_ref, Wx_ref, bx_ref, Wr_ref,
          br_ref, Wkn_ref, bkn_ref, g_ref, beta_ref, out_ref, h_scr, px_scr):
    s = pl.program_id(0)

    @pl.when(s == 0)
    def _know_prologue():
        pk = jnp.dot(ck_ref[...], Wkn_ref[...],
                     preferred_element_type=jnp.float32) + bkn_ref[...]
        _ln_into(h_scr, 7, pk, g_ref, beta_ref)

    @pl.when(s == 1)
    def _attn_prologue():
        pr = jnp.dot(ca_ref[...], Wr_ref[...],
                     preferred_element_type=jnp.float32) + br_ref[...]
        for k in range(3):
            _ln_into(h_scr, 4 + k, pr[:, k * D_SPACE:(k + 1) * D_SPACE],
                     g_ref, beta_ref)

    @pl.when(s == 2)
    def _x_prologue_a():
        px_scr[...] = jnp.dot(x_ref[...], Wx_ref[0],
                              preferred_element_type=jnp.float32)

    @pl.when(s == 3)
    def _x_prologue_b():
        px = px_scr[...] + jnp.dot(x_ref[...], Wx_ref[1],
                                   preferred_element_type=jnp.float32)
        px = px + bx_ref[...]
        for k in range(4):
            _ln_into(h_scr, k, px[:, k * D_SPACE:(k + 1) * D_SPACE],
                     g_ref, beta_ref)

    e = ne_ref[...].astype(jnp.float32)
    inv = 1.0 / jnp.maximum(
        jnp.sqrt(jnp.sum(e * e, axis=-1, keepdims=True)), 1e-12)
    en = (e * inv).astype(jnp.bfloat16)
    h = h_scr[tab_ref[2, s]]
    out_ref[...] = jax.lax.dot_general(
        h, en, (((1,), (1,)), ((), ())), preferred_element_type=jnp.float32)


def kernel(x, ctx_attn, ctx_know, neuron_emb, W_feat, b_feat, W_know, b_know,
           W_rQ, b_rQ, W_rK, b_rK, W_rV, b_rV, W_rKn, b_rKn,
           g_fqkQ, beta_fqkQ, g_fqkK, beta_fqkK, g_fv, beta_fv,
           g_fkn, beta_fkn, g_rQ, beta_rQ, g_rK, beta_rK,
           g_rV, beta_rV, g_rKn, beta_rKn):
    B = x.shape[0]
    x2 = x.reshape(B * S, D_MODEL).astype(jnp.bfloat16)
    ca = ctx_attn.reshape(B * S, -1).astype(jnp.bfloat16)
    ck = ctx_know.reshape(B * S, -1).astype(jnp.bfloat16)

    Wx = jnp.concatenate([W_feat, W_know], axis=1)
    Wxs = Wx.astype(jnp.bfloat16).reshape(2, XK, 256)
    bx = jnp.concatenate([b_feat, b_know])[None, :]
    Wr = jnp.concatenate([W_rQ, W_rK, W_rV], axis=1).astype(jnp.bfloat16)
    br = jnp.concatenate([b_rQ, b_rK, b_rV])[None, :]
    Wkn = W_rKn.astype(jnp.bfloat16)
    bkn = b_rKn[None, :]
    g = jnp.concatenate([g_fqkQ, g_fqkK, g_fv, g_fkn,
                         g_rQ, g_rK, g_rV, g_rKn])[None, :]
    beta = jnp.concatenate([beta_fqkQ, beta_fqkK, beta_fv, beta_fkn,
                            beta_rQ, beta_rK, beta_rV, beta_rKn])[None, :]

    tab = jnp.asarray([_NTAB, _OTAB, _HTAB], dtype=jnp.int32)
    full = lambda a: pl.BlockSpec(a.shape, lambda s, t: (0,) * a.ndim)

    grid_spec = pltpu.PrefetchScalarGridSpec(
        num_scalar_prefetch=1,
        grid=(NUM_J,),
        in_specs=[
            pl.BlockSpec((B * S, XK),
                         lambda s, t: (0, jnp.clip(s - 2, 0, 1))),
            full(ca), full(ck),
            pl.BlockSpec((TN, D_SPACE), lambda s, t: (t[0, s], 0)),
            full(Wxs), full(bx), full(Wr), full(br),
            full(Wkn), full(bkn), full(g), full(beta),
        ],
        out_specs=pl.BlockSpec((B * S, TN), lambda s, t: (0, t[1, s])),
        scratch_shapes=[pltpu.VMEM((8, B * S, D_SPACE), jnp.bfloat16),
                        pltpu.VMEM((B * S, 256), jnp.float32)],
    )

    out = pl.pallas_call(
        _body,
        grid_spec=grid_spec,
        out_shape=jax.ShapeDtypeStruct((B * S, N_OUT), jnp.float32),
    )(tab, x2, ca, ck, neuron_emb.astype(jnp.bfloat16),
      Wxs, bx, Wr, br, Wkn, bkn, g, beta)

    return out.reshape(B, S, N_OUT)


# speedup vs baseline: 1.1826x; 1.0598x over previous
import jax
import jax.numpy as jnp
from jax.experimental import pallas as pl
from jax.experimental.pallas import tpu as pltpu

D_MODEL = 1024
D_SPACE = 64
S = 2048
N_OUT = 20480
TN = 1024
NUM_J = N_OUT // TN
XK = D_MODEL // 2

_SEGS = (
    (7, 12, 4, 16),
    (4, 4, 2, 10),
    (5, 4, 2, 12),
    (6, 6, 2, 14),
    (0, 0, 2, 0),
    (1, 0, 2, 2),
    (2, 2, 2, 4),
    (3, 8, 4, 6),
)
_HTAB = tuple(h for h, n0, nn, o0 in _SEGS for _ in range(nn))
_NTAB = tuple(n0 + k for h, n0, nn, o0 in _SEGS for k in range(nn))
_OTAB = tuple(o0 + k for h, n0, nn, o0 in _SEGS for k in range(nn))


def _group_mean_mat(n):
    r = jax.lax.broadcasted_iota(jnp.int32, (n, n), 0) // D_SPACE
    c = jax.lax.broadcasted_iota(jnp.int32, (n, n), 1) // D_SPACE
    return jnp.where(r == c, 1.0 / D_SPACE, 0.0).astype(jnp.float32)


def _ln_heads(scr, k0, t, g_ref, b_ref):
    n = t.shape[-1]
    gm = _group_mean_mat(n)
    m = jnp.dot(t, gm, preferred_element_type=jnp.float32)
    ms = jnp.dot(t * t, gm, preferred_element_type=jnp.float32)
    v = ms - m * m
    g = g_ref[:, k0 * D_SPACE:k0 * D_SPACE + n]
    b = b_ref[:, k0 * D_SPACE:k0 * D_SPACE + n]
    h = ((t - m) * jax.lax.rsqrt(v + 1e-5) * g + b).astype(jnp.bfloat16)
    for k in range(n // D_SPACE):
        scr[k0 + k] = h[:, k * D_SPACE:(k + 1) * D_SPACE]


def _body(tab_ref, x_ref, ca_ref, ck_ref, ne_ref, Wx_ref, bx_ref, Wr_ref,
          br_ref, Wkn_ref, bkn_ref, g_ref, beta_ref, out_ref, h_scr, px_scr):
    s = pl.program_id(0)

    @pl.when(s == 0)
    def _know_prologue():
        pk = jnp.dot(ck_ref[...], Wkn_ref[...],
                     preferred_element_type=jnp.float32) + bkn_ref[...]
        _ln_heads(h_scr, 7, pk, g_ref, beta_ref)

    @pl.when(s == 1)
    def _attn_prologue():
        pr = jnp.dot(ca_ref[...], Wr_ref[...],
                     preferred_element_type=jnp.float32) + br_ref[...]
        _ln_heads(h_scr, 4, pr, g_ref, beta_ref)

    @pl.when(s == 2)
    def _x_prologue_a():
        px_scr[...] = jnp.dot(x_ref[...], Wx_ref[0],
                              preferred_element_type=jnp.float32)

    @pl.when(s == 3)
    def _x_prologue_b():
        px = px_scr[...] + jnp.dot(x_ref[...], Wx_ref[1],
                                   preferred_element_type=jnp.float32)
        px = px + bx_ref[...]
        _ln_heads(h_scr, 0, px, g_ref, beta_ref)

    e = ne_ref[...].astype(jnp.float32)
    s2 = jnp.dot(e * e, _group_mean_mat(D_SPACE) * D_SPACE,
                 preferred_element_type=jnp.float32)
    inv = 1.0 / jnp.maximum(jnp.sqrt(s2), 1e-12)
    en = (e * inv).astype(jnp.bfloat16)
    h = h_scr[tab_ref[2, s]]
    out_ref[...] = jax.lax.dot_general(
        h, en, (((1,), (1,)), ((), ())), preferred_element_type=jnp.float32)


def kernel(x, ctx_attn, ctx_know, neuron_emb, W_feat, b_feat, W_know, b_know,
           W_rQ, b_rQ, W_rK, b_rK, W_rV, b_rV, W_rKn, b_rKn,
           g_fqkQ, beta_fqkQ, g_fqkK, beta_fqkK, g_fv, beta_fv,
           g_fkn, beta_fkn, g_rQ, beta_rQ, g_rK, beta_rK,
           g_rV, beta_rV, g_rKn, beta_rKn):
    B = x.shape[0]
    x2 = x.reshape(B * S, D_MODEL).astype(jnp.bfloat16)
    ca = ctx_attn.reshape(B * S, -1).astype(jnp.bfloat16)
    ck = ctx_know.reshape(B * S, -1).astype(jnp.bfloat16)

    Wx = jnp.concatenate([W_feat, W_know], axis=1)
    Wxs = Wx.astype(jnp.bfloat16).reshape(2, XK, 256)
    bx = jnp.concatenate([b_feat, b_know])[None, :]
    Wr = jnp.concatenate([W_rQ, W_rK, W_rV], axis=1).astype(jnp.bfloat16)
    br = jnp.concatenate([b_rQ, b_rK, b_rV])[None, :]
    Wkn = W_rKn.astype(jnp.bfloat16)
    bkn = b_rKn[None, :]
    g = jnp.concatenate([g_fqkQ, g_fqkK, g_fv, g_fkn,
                         g_rQ, g_rK, g_rV, g_rKn])[None, :]
    beta = jnp.concatenate([beta_fqkQ, beta_fqkK, beta_fv, beta_fkn,
                            beta_rQ, beta_rK, beta_rV, beta_rKn])[None, :]

    tab = jnp.asarray([_NTAB, _OTAB, _HTAB], dtype=jnp.int32)
    full = lambda a: pl.BlockSpec(a.shape, lambda s, t: (0,) * a.ndim)

    grid_spec = pltpu.PrefetchScalarGridSpec(
        num_scalar_prefetch=1,
        grid=(NUM_J,),
        in_specs=[
            pl.BlockSpec((B * S, XK),
                         lambda s, t: (0, jnp.clip(s - 2, 0, 1))),
            full(ca), full(ck),
            pl.BlockSpec((TN, D_SPACE), lambda s, t: (t[0, s], 0)),
            full(Wxs), full(bx), full(Wr), full(br),
            full(Wkn), full(bkn), full(g), full(beta),
        ],
        out_specs=pl.BlockSpec((B * S, TN), lambda s, t: (0, t[1, s])),
        scratch_shapes=[pltpu.VMEM((8, B * S, D_SPACE), jnp.bfloat16),
                        pltpu.VMEM((B * S, 256), jnp.float32)],
    )

    out = pl.pallas_call(
        _body,
        grid_spec=grid_spec,
        out_shape=jax.ShapeDtypeStruct((B * S, N_OUT), jnp.float32),
    )(tab, x2, ca, ck, neuron_emb.astype(jnp.bfloat16),
      Wxs, bx, Wr, br, Wkn, bkn, g, beta)

    return out.reshape(B, S, N_OUT)


# speedup vs baseline: 1.1897x; 1.0060x over previous
import jax
import jax.numpy as jnp
from jax.experimental import pallas as pl
from jax.experimental.pallas import tpu as pltpu

D_MODEL = 1024
D_SPACE = 64
S = 2048
N_OUT = 20480
TN = 1024
NUM_J = N_OUT // TN
XK = D_MODEL // 2

_SEGS = (
    (7, 12, 4, 16),
    (4, 4, 2, 10),
    (5, 4, 2, 12),
    (6, 6, 2, 14),
    (0, 0, 2, 0),
    (1, 0, 2, 2),
    (2, 2, 2, 4),
    (3, 8, 4, 6),
)
_HTAB = tuple(h for h, n0, nn, o0 in _SEGS for _ in range(nn))
_NTAB = tuple(n0 + k for h, n0, nn, o0 in _SEGS for k in range(nn))
_OTAB = tuple(o0 + k for h, n0, nn, o0 in _SEGS for k in range(nn))


def _group_mean_mat(n):
    r = jax.lax.broadcasted_iota(jnp.int32, (n, n), 0) // D_SPACE
    c = jax.lax.broadcasted_iota(jnp.int32, (n, n), 1) // D_SPACE
    return jnp.where(r == c, 1.0 / D_SPACE, 0.0).astype(jnp.float32)


def _ln_heads(scr, k0, t, g_ref, b_ref):
    n = t.shape[-1]
    gm = _group_mean_mat(n)
    m = jnp.dot(t, gm, preferred_element_type=jnp.float32)
    ms = jnp.dot(t * t, gm, preferred_element_type=jnp.float32)
    v = ms - m * m
    g = g_ref[:, k0 * D_SPACE:k0 * D_SPACE + n]
    b = b_ref[:, k0 * D_SPACE:k0 * D_SPACE + n]
    h = ((t - m) * jax.lax.rsqrt(v + 1e-5) * g + b).astype(jnp.bfloat16)
    for k in range(n // D_SPACE):
        scr[k0 + k] = h[:, k * D_SPACE:(k + 1) * D_SPACE]


def _body(tab_ref, x_ref, ca_ref, ck_ref, ne_ref, Wx_ref, bx_ref, Wr_ref,
          br_ref, Wkn_ref, bkn_ref, g_ref, beta_ref, out_ref, h_scr, px_scr):
    s = pl.program_id(0)

    @pl.when(s == 0)
    def _know_prologue():
        pk = jnp.dot(ck_ref[...].astype(jnp.bfloat16), Wkn_ref[...],
                     preferred_element_type=jnp.float32) + bkn_ref[...]
        _ln_heads(h_scr, 7, pk, g_ref, beta_ref)

    @pl.when(s == 1)
    def _attn_prologue():
        pr = jnp.dot(ca_ref[...].astype(jnp.bfloat16), Wr_ref[...],
                     preferred_element_type=jnp.float32) + br_ref[...]
        _ln_heads(h_scr, 4, pr, g_ref, beta_ref)

    @pl.when(s == 2)
    def _x_prologue_a():
        px_scr[...] = jnp.dot(x_ref[...].astype(jnp.bfloat16), Wx_ref[0],
                              preferred_element_type=jnp.float32)

    @pl.when(s == 3)
    def _x_prologue_b():
        px = px_scr[...] + jnp.dot(x_ref[...].astype(jnp.bfloat16), Wx_ref[1],
                                   preferred_element_type=jnp.float32)
        px = px + bx_ref[...]
        _ln_heads(h_scr, 0, px, g_ref, beta_ref)

    e = ne_ref[...]
    s2 = jnp.dot(e * e, _group_mean_mat(D_SPACE) * D_SPACE,
                 preferred_element_type=jnp.float32)
    inv = 1.0 / jnp.maximum(jnp.sqrt(s2), 1e-12)
    en = (e * inv).astype(jnp.bfloat16)
    h = h_scr[tab_ref[2, s]]
    out_ref[...] = jax.lax.dot_general(
        h, en, (((1,), (1,)), ((), ())), preferred_element_type=jnp.float32)


def kernel(x, ctx_attn, ctx_know, neuron_emb, W_feat, b_feat, W_know, b_know,
           W_rQ, b_rQ, W_rK, b_rK, W_rV, b_rV, W_rKn, b_rKn,
           g_fqkQ, beta_fqkQ, g_fqkK, beta_fqkK, g_fv, beta_fv,
           g_fkn, beta_fkn, g_rQ, beta_rQ, g_rK, beta_rK,
           g_rV, beta_rV, g_rKn, beta_rKn):
    B = x.shape[0]
    x2 = x.reshape(B * S, D_MODEL)
    ca = ctx_attn.reshape(B * S, -1)
    ck = ctx_know.reshape(B * S, -1)

    Wx = jnp.concatenate([W_feat, W_know], axis=1)
    Wxs = Wx.astype(jnp.bfloat16).reshape(2, XK, 256)
    bx = jnp.concatenate([b_feat, b_know])[None, :]
    Wr = jnp.concatenate([W_rQ, W_rK, W_rV], axis=1).astype(jnp.bfloat16)
    br = jnp.concatenate([b_rQ, b_rK, b_rV])[None, :]
    Wkn = W_rKn.astype(jnp.bfloat16)
    bkn = b_rKn[None, :]
    g = jnp.concatenate([g_fqkQ, g_fqkK, g_fv, g_fkn,
                         g_rQ, g_rK, g_rV, g_rKn])[None, :]
    beta = jnp.concatenate([beta_fqkQ, beta_fqkK, beta_fv, beta_fkn,
                            beta_rQ, beta_rK, beta_rV, beta_rKn])[None, :]

    tab = jnp.asarray([_NTAB, _OTAB, _HTAB], dtype=jnp.int32)
    full = lambda a: pl.BlockSpec(a.shape, lambda s, t: (0,) * a.ndim)

    grid_spec = pltpu.PrefetchScalarGridSpec(
        num_scalar_prefetch=1,
        grid=(NUM_J,),
        in_specs=[
            pl.BlockSpec((B * S, XK),
                         lambda s, t: (0, jnp.clip(s - 2, 0, 1))),
            full(ca), full(ck),
            pl.BlockSpec((TN, D_SPACE), lambda s, t: (t[0, s], 0)),
            full(Wxs), full(bx), full(Wr), full(br),
            full(Wkn), full(bkn), full(g), full(beta),
        ],
        out_specs=pl.BlockSpec((B * S, TN), lambda s, t: (0, t[1, s])),
        scratch_shapes=[pltpu.VMEM((8, B * S, D_SPACE), jnp.bfloat16),
                        pltpu.VMEM((B * S, 256), jnp.float32)],
    )

    out = pl.pallas_call(
        _body,
        grid_spec=grid_spec,
        out_shape=jax.ShapeDtypeStruct((B * S, N_OUT), jnp.float32),
    )(tab, x2, ca, ck, neuron_emb, Wxs, bx, Wr, br, Wkn, bkn, g, beta)

    return out.reshape(B, S, N_OUT)
